# Initial kernel scaffold; baseline (speedup 1.0000x reference)
import jax
import jax.numpy as jnp
from jax.experimental import pallas as pl


def kernel(coords_rec, h_feats_rec, original_receptor_node_features, edge_feat, We1, be1, ge, bne, We2, be2, Wn1, bn1, gn, bnn, Wn2, bn2, gf, bf, edge_index):
    raise NotImplementedError("write your pallas kernel here")


# speedup vs baseline: 2.2383x; 2.2383x over previous
import functools

import jax
import jax.numpy as jnp
from jax import lax
from jax.experimental import pallas as pl
from jax.experimental.pallas import tpu as pltpu
from jax.experimental.pallas import tpu_sc as plsc

_N = 10000
_E = 320000
_D = 128
_DE = 16
_SKIPW = 0.5
_EPS = 1e-5

_NC = 2
_NS = 16
_EW = _E // (_NC * _NS)
_CHUNK = 80
_NCH = _EW // _CHUNK
_NP = 10240
_RPT = _NP // _NS
_RCH = _RPT // _CHUNK
_CR = _NP // _D


def _ln_tc(x, g, b):
    mu = jnp.mean(x, axis=-1, keepdims=True)
    var = jnp.mean((x - mu) ** 2, axis=-1, keepdims=True)
    return (x - mu) * lax.rsqrt(var + _EPS) * g + b


def _pq_body(h_ref, wa_ref, wb_ref, p_ref, q_ref):
    h = h_ref[...]
    p_ref[...] = jnp.dot(h, wa_ref[...], preferred_element_type=jnp.float32)
    q_ref[...] = jnp.dot(h, wb_ref[...], preferred_element_type=jnp.float32)


def _pq(h, wa, wb):
    nb = 1000
    return pl.pallas_call(
        _pq_body,
        grid=(_N // nb,),
        in_specs=[
            pl.BlockSpec((nb, _D), lambda i: (i, 0)),
            pl.BlockSpec((_D, _D), lambda i: (0, 0)),
            pl.BlockSpec((_D, _D), lambda i: (0, 0)),
        ],
        out_specs=[
            pl.BlockSpec((nb, _D), lambda i: (i, 0)),
            pl.BlockSpec((nb, _D), lambda i: (i, 0)),
        ],
        out_shape=[
            jax.ShapeDtypeStruct((_N, _D), jnp.float32),
            jax.ShapeDtypeStruct((_N, _D), jnp.float32),
        ],
    )(h, wa, wb)


def _r_body(ef_ref, wc_ref, be1_ref, r_ref):
    r_ref[...] = (
        jnp.dot(ef_ref[...], wc_ref[...], preferred_element_type=jnp.float32)
        + be1_ref[...]
    )


def _redge(ef, wc, be1_row):
    eb = 8000
    return pl.pallas_call(
        _r_body,
        grid=(_E // eb,),
        in_specs=[
            pl.BlockSpec((eb, _DE), lambda i: (i, 0)),
            pl.BlockSpec((_DE, _D), lambda i: (0, 0)),
            pl.BlockSpec((1, _D), lambda i: (0, 0)),
        ],
        out_specs=pl.BlockSpec((eb, _D), lambda i: (i, 0)),
        out_shape=jax.ShapeDtypeStruct((_E, _D), jnp.float32),
    )(ef, wc, be1_row)


def _sc_edge_body(p_hbm, q_hbm, r_hbm, src_hbm, dst_hbm, ge_hbm, bne_hbm,
                  out_hbm, cnt_hbm, acc, cntacc, src_v, dst_v, ps_v, qd_v,
                  u_v, cnt_v, rowidx_v, ge_v, bne_v, sem1, sem2):
    cc = lax.axis_index("c")
    ss = lax.axis_index("s")
    zero16 = jnp.zeros((16,), jnp.float32)
    one16 = jnp.full((16,), 1.0, jnp.float32)
    lane = lax.iota(jnp.int32, 16)
    cols = [jnp.full((16,), 16 * j, jnp.int32) + lane for j in range(_D // 16)]

    pltpu.sync_copy(ge_hbm, ge_v)
    pltpu.sync_copy(bne_hbm, bne_v)

    def _zrow(i, carry):
        rows = jnp.full((16,), i, jnp.int32)
        for j in range(_D // 16):
            plsc.store_scatter(u_v, [rows, cols[j]], zero16)
        return carry

    lax.fori_loop(0, _CHUNK, _zrow, 0)

    def _zcnt(i, carry):
        rows = jnp.full((16,), i, jnp.int32)
        for j in range(_D // 16):
            plsc.store_scatter(cnt_v, [rows, cols[j]], zero16)
        return carry

    lax.fori_loop(0, _CR, _zcnt, 0)
    for k in range(_CHUNK // 16):
        rowidx_v[pl.ds(16 * k, 16)] = jnp.full((16,), 16 * k, jnp.int32) + lane

    nbase = ss * _RPT
    for k in range(_RCH):
        pltpu.sync_copy(u_v, acc.at[pl.ds(nbase + k * _CHUNK, _CHUNK), :])

    @pl.when(ss == 0)
    def _():
        pltpu.sync_copy(u_v, cntacc)

    plsc.subcore_barrier()

    gs = [ge_v[pl.ds(16 * j, 16)] for j in range(_D // 16)]
    bs = [bne_v[pl.ds(16 * j, 16)] for j in range(_D // 16)]
    ebase = (cc * _NS + ss) * _EW

    def _chunk(ci, carry):
        base = ebase + ci * _CHUNK
        pltpu.sync_copy(src_hbm.at[pl.ds(base, _CHUNK)], src_v)
        pltpu.sync_copy(dst_hbm.at[pl.ds(base, _CHUNK)], dst_v)
        cp1 = pltpu.async_copy(p_hbm.at[src_v], ps_v, sem1)
        cp2 = pltpu.async_copy(q_hbm.at[dst_v], qd_v, sem2)
        pltpu.sync_copy(r_hbm.at[pl.ds(base, _CHUNK), :], u_v)
        cp1.wait()
        cp2.wait()

        for k in range(_CHUNK // 16):
            idx = dst_v[pl.ds(16 * k, 16)]
            plsc.addupdate_scatter(cnt_v, [idx >> 7, idx & 127], one16)

        def _edge(e, c2):
            rows = jnp.full((16,), e, jnp.int32)
            vs = [
                jnp.maximum(
                    plsc.load_gather(ps_v, [rows, cols[j]])
                    + plsc.load_gather(qd_v, [rows, cols[j]])
                    + plsc.load_gather(u_v, [rows, cols[j]]),
                    zero16,
                )
                for j in range(_D // 16)
            ]
            tot = vs[0]
            for j in range(1, _D // 16):
                tot = tot + vs[j]
            mean = jnp.sum(tot) * (1.0 / _D)
            mv = jnp.full((16,), mean)
            dv = [v - mv for v in vs]
            sq = dv[0] * dv[0]
            for j in range(1, _D // 16):
                sq = sq + dv[j] * dv[j]
            var = jnp.sum(sq) * (1.0 / _D)
            xv = jnp.full((16,), var + _EPS)
            xi = plsc.bitcast(xv, jnp.int32)
            y = plsc.bitcast(jnp.full((16,), 0x5F3759DF, jnp.int32) - (xi >> 1),
                             jnp.float32)
            c15 = jnp.full((16,), 1.5, jnp.float32)
            ch = jnp.full((16,), 0.5, jnp.float32)
            for _ in range(3):
                y = y * (c15 - ch * xv * y * y)
            for j in range(_D // 16):
                plsc.store_scatter(u_v, [rows, cols[j]], dv[j] * y * gs[j] + bs[j])
            return c2

        lax.fori_loop(0, _CHUNK, _edge, 0)
        pltpu.sync_copy(u_v, acc.at[dst_v], add=True)
        return carry

    lax.fori_loop(0, _NCH, _chunk, 0)

    pltpu.sync_copy(cnt_v, cntacc.at[rowidx_v], add=True)
    plsc.subcore_barrier()

    for k in range(_RCH):
        pltpu.sync_copy(acc.at[pl.ds(nbase + k * _CHUNK, _CHUNK), :], u_v)
        pltpu.sync_copy(u_v, out_hbm.at[cc, pl.ds(nbase + k * _CHUNK, _CHUNK), :])

    @pl.when(ss == 0)
    def _():
        pltpu.sync_copy(cntacc, cnt_hbm.at[cc])


def _sc_edge(p, q, r, src, dst, ge, bne):
    mesh = plsc.VectorSubcoreMesh(
        core_axis_name="c", subcore_axis_name="s",
        num_cores=_NC, num_subcores=_NS,
    )
    fn = functools.partial(
        pl.kernel,
        compiler_params=pltpu.CompilerParams(needs_layout_passes=False),
        out_type=[
            jax.ShapeDtypeStruct((_NC, _NP, _D), jnp.float32),
            jax.ShapeDtypeStruct((_NC, _CR, _D), jnp.float32),
        ],
        mesh=mesh,
        scratch_types=[
            pltpu.VMEM_SHARED((_NP, _D), jnp.float32),
            pltpu.VMEM_SHARED((_CR, _D), jnp.float32),
            pltpu.VMEM((_CHUNK,), jnp.int32),
            pltpu.VMEM((_CHUNK,), jnp.int32),
            pltpu.VMEM((_CHUNK, _D), jnp.float32),
            pltpu.VMEM((_CHUNK, _D), jnp.float32),
            pltpu.VMEM((_CHUNK, _D), jnp.float32),
            pltpu.VMEM((_CR, _D), jnp.float32),
            pltpu.VMEM((_CHUNK,), jnp.int32),
            pltpu.VMEM((_D,), jnp.float32),
            pltpu.VMEM((_D,), jnp.float32),
            pltpu.SemaphoreType.DMA,
            pltpu.SemaphoreType.DMA,
        ],
    )(_sc_edge_body)
    return fn(p, q, r, src, dst, ge, bne)


def _node_body(part_ref, cnt_ref, h_ref, orig_ref, we2_ref, wa_ref, wb_ref,
               wc_ref, wn2_ref, be2_ref, bn1_ref, gn_ref, bnn_ref, bn2_ref,
               gf_ref, bf_ref, out_ref):
    asum = part_ref[0] + part_ref[1]
    cnt = cnt_ref[0] + cnt_ref[1]
    inv = 1.0 / jnp.maximum(cnt, 1.0)
    flag = jnp.minimum(cnt, 1.0)
    aggr = (
        jnp.dot(asum * inv, we2_ref[...], preferred_element_type=jnp.float32)
        + flag * be2_ref[...]
    )
    h = h_ref[...]
    x = (
        jnp.dot(h, wa_ref[...], preferred_element_type=jnp.float32)
        + jnp.dot(aggr, wb_ref[...], preferred_element_type=jnp.float32)
        + jnp.dot(orig_ref[...], wc_ref[...], preferred_element_type=jnp.float32)
        + bn1_ref[...]
    )
    x = jnp.maximum(x, 0.0)
    x = _ln_tc(x, gn_ref[...], bnn_ref[...])
    upd = jnp.dot(x, wn2_ref[...], preferred_element_type=jnp.float32) + bn2_ref[...]
    out = _SKIPW * upd + (1.0 - _SKIPW) * h
    out_ref[...] = _ln_tc(out, gf_ref[...], bf_ref[...])


def _node(part, cnt3, h, orig, we2, wn1a, wn1b, wn1c, wn2, rows):
    nb = 1000
    mat = lambda: pl.BlockSpec((_D, _D), lambda i: (0, 0))
    vec = lambda: pl.BlockSpec((1, _D), lambda i: (0, 0))
    return pl.pallas_call(
        _node_body,
        grid=(_N // nb,),
        in_specs=[
            pl.BlockSpec((_NC, nb, _D), lambda i: (0, i, 0)),
            pl.BlockSpec((_NC, nb, 1), lambda i: (0, i, 0)),
            pl.BlockSpec((nb, _D), lambda i: (i, 0)),
            pl.BlockSpec((nb, _D), lambda i: (i, 0)),
            mat(), mat(), mat(), mat(), mat(),
            vec(), vec(), vec(), vec(), vec(), vec(), vec(),
        ],
        out_specs=pl.BlockSpec((nb, _D), lambda i: (i, 0)),
        out_shape=jax.ShapeDtypeStruct((_N, _D), jnp.float32),
    )(part, cnt3, h, orig, we2, wn1a, wn1b, wn1c, wn2, *rows)


def kernel(coords_rec, h_feats_rec, original_receptor_node_features, edge_feat,
           We1, be1, ge, bne, We2, be2, Wn1, bn1, gn, bnn, Wn2, bn2, gf, bf,
           edge_index):
    del coords_rec
    src = edge_index[0].astype(jnp.int32)
    dst = edge_index[1].astype(jnp.int32)
    w1a = We1[:_D]
    w1b = We1[_D:2 * _D]
    w1c = We1[2 * _D:]

    p, q = _pq(h_feats_rec, w1a, w1b)
    r = _redge(edge_feat, w1c, be1.reshape(1, _D))
    part, cnt = _sc_edge(p, q, r, src, dst, ge, bne)
    cnt3 = cnt.reshape(_NC, _NP, 1)

    rows = [v.reshape(1, _D) for v in (be2, bn1, gn, bnn, bn2, gf, bf)]
    return _node(part, cnt3, h_feats_rec, original_receptor_node_features,
                 We2, Wn1[:_D], Wn1[_D:2 * _D], Wn1[2 * _D:], Wn2, rows)


# speedup vs baseline: 2.5481x; 1.1384x over previous
import functools

import jax
import jax.numpy as jnp
from jax import lax
from jax.experimental import pallas as pl
from jax.experimental.pallas import tpu as pltpu
from jax.experimental.pallas import tpu_sc as plsc

_N = 10000
_E = 320000
_D = 128
_DE = 16
_SKIPW = 0.5
_EPS = 1e-5

_NC = 2
_NS = 16
_EW = _E // (_NC * _NS)
_CHUNK = 80
_NCH = _EW // _CHUNK
_NP = 10240
_RPT = _NP // _NS
_RCH = _RPT // _CHUNK
_CR = _NP // _D


def _ln_tc(x, g, b):
    mu = jnp.mean(x, axis=-1, keepdims=True)
    var = jnp.mean((x - mu) ** 2, axis=-1, keepdims=True)
    return (x - mu) * lax.rsqrt(var + _EPS) * g + b


def _pq_body(h_ref, wa_ref, wb_ref, p_ref, q_ref):
    h = h_ref[...]
    p_ref[...] = jnp.dot(h, wa_ref[...], preferred_element_type=jnp.float32)
    q_ref[...] = jnp.dot(h, wb_ref[...], preferred_element_type=jnp.float32)


def _pq(h, wa, wb):
    nb = 1000
    return pl.pallas_call(
        _pq_body,
        grid=(_N // nb,),
        in_specs=[
            pl.BlockSpec((nb, _D), lambda i: (i, 0)),
            pl.BlockSpec((_D, _D), lambda i: (0, 0)),
            pl.BlockSpec((_D, _D), lambda i: (0, 0)),
        ],
        out_specs=[
            pl.BlockSpec((nb, _D), lambda i: (i, 0)),
            pl.BlockSpec((nb, _D), lambda i: (i, 0)),
        ],
        out_shape=[
            jax.ShapeDtypeStruct((_N, _D), jnp.float32),
            jax.ShapeDtypeStruct((_N, _D), jnp.float32),
        ],
    )(h, wa, wb)


def _r_body(ef_ref, wc_ref, be1_ref, r_ref):
    r_ref[...] = (
        jnp.dot(ef_ref[...], wc_ref[...], preferred_element_type=jnp.float32)
        + be1_ref[...]
    )


def _redge(ef, wc, be1_row):
    eb = 8000
    return pl.pallas_call(
        _r_body,
        grid=(_E // eb,),
        in_specs=[
            pl.BlockSpec((eb, _DE), lambda i: (i, 0)),
            pl.BlockSpec((_DE, _D), lambda i: (0, 0)),
            pl.BlockSpec((1, _D), lambda i: (0, 0)),
        ],
        out_specs=pl.BlockSpec((eb, _D), lambda i: (i, 0)),
        out_shape=jax.ShapeDtypeStruct((_E, _D), jnp.float32),
    )(ef, wc, be1_row)


def _sc_edge_body(p_hbm, q_hbm, r_hbm, src_hbm, dst_hbm, ge_hbm, bne_hbm,
                  out_hbm, cnt_hbm, acc, cntacc, src_v, dst_v, ps_v, qd_v,
                  u_v, cnt_v, rowidx_v, ge_v, bne_v, sem1, sem2):
    cc = lax.axis_index("c")
    ss = lax.axis_index("s")
    zero16 = jnp.zeros((16,), jnp.float32)
    one16 = jnp.full((16,), 1.0, jnp.float32)
    lane = lax.iota(jnp.int32, 16)
    cols = [jnp.full((16,), 16 * j, jnp.int32) + lane for j in range(_D // 16)]

    pltpu.sync_copy(ge_hbm, ge_v)
    pltpu.sync_copy(bne_hbm, bne_v)

    def _zrow(i, carry):
        rows = jnp.full((16,), i, jnp.int32)
        for j in range(_D // 16):
            plsc.store_scatter(u_v, [rows, cols[j]], zero16)
        return carry

    lax.fori_loop(0, _CHUNK, _zrow, 0)

    def _zcnt(i, carry):
        rows = jnp.full((16,), i, jnp.int32)
        for j in range(_D // 16):
            plsc.store_scatter(cnt_v, [rows, cols[j]], zero16)
        return carry

    lax.fori_loop(0, _CR, _zcnt, 0)
    for k in range(_CHUNK // 16):
        rowidx_v[pl.ds(16 * k, 16)] = jnp.full((16,), 16 * k, jnp.int32) + lane

    nbase = ss * _RPT
    for k in range(_RCH):
        pltpu.sync_copy(u_v, acc.at[pl.ds(nbase + k * _CHUNK, _CHUNK), :])

    @pl.when(ss == 0)
    def _():
        pltpu.sync_copy(u_v, cntacc)

    plsc.subcore_barrier()

    gs = [ge_v[pl.ds(16 * j, 16)] for j in range(_D // 16)]
    bs = [bne_v[pl.ds(16 * j, 16)] for j in range(_D // 16)]
    ebase = (cc * _NS + ss) * _EW

    def _chunk(ci, carry):
        base = ebase + ci * _CHUNK
        cpa = pltpu.async_copy(src_hbm.at[pl.ds(base, _CHUNK)], src_v, sem1)
        cpb = pltpu.async_copy(dst_hbm.at[pl.ds(base, _CHUNK)], dst_v, sem2)
        cpa.wait()
        cpb.wait()
        cp1 = pltpu.async_copy(p_hbm.at[src_v], ps_v, sem1)
        cp2 = pltpu.async_copy(q_hbm.at[dst_v], qd_v, sem2)
        pltpu.sync_copy(r_hbm.at[pl.ds(base, _CHUNK), :], u_v)
        cp1.wait()
        cp2.wait()

        for k in range(_CHUNK // 16):
            idx = dst_v[pl.ds(16 * k, 16)]
            plsc.addupdate_scatter(cnt_v, [idx >> 7, idx & 127], one16)

        @plsc.parallel_loop(0, _CHUNK, 1, unroll=4)
        def _edge(e):
            rows = jnp.full((16,), e, jnp.int32)
            vs = [
                jnp.maximum(
                    plsc.load_gather(ps_v, [rows, cols[j]])
                    + plsc.load_gather(qd_v, [rows, cols[j]])
                    + plsc.load_gather(u_v, [rows, cols[j]]),
                    zero16,
                )
                for j in range(_D // 16)
            ]
            tot = vs[0]
            sq = vs[0] * vs[0]
            for j in range(1, _D // 16):
                tot = tot + vs[j]
                sq = sq + vs[j] * vs[j]
            mean = jnp.sum(tot) * (1.0 / _D)
            ex2 = jnp.sum(sq) * (1.0 / _D)
            var = jnp.maximum(ex2 - mean * mean, 0.0)
            xv = jnp.full((16,), var + _EPS)
            mv = jnp.full((16,), mean)
            xi = plsc.bitcast(xv, jnp.int32)
            y = plsc.bitcast(jnp.full((16,), 0x5F3759DF, jnp.int32) - (xi >> 1),
                             jnp.float32)
            c15 = jnp.full((16,), 1.5, jnp.float32)
            ch = jnp.full((16,), 0.5, jnp.float32)
            for _ in range(3):
                y = y * (c15 - ch * xv * y * y)
            for j in range(_D // 16):
                plsc.store_scatter(u_v, [rows, cols[j]],
                                   (vs[j] - mv) * y * gs[j] + bs[j])

        pltpu.sync_copy(u_v, acc.at[dst_v], add=True)
        return carry

    lax.fori_loop(0, _NCH, _chunk, 0)

    pltpu.sync_copy(cnt_v, cntacc.at[rowidx_v], add=True)
    plsc.subcore_barrier()

    for k in range(_RCH):
        pltpu.sync_copy(acc.at[pl.ds(nbase + k * _CHUNK, _CHUNK), :], u_v)
        pltpu.sync_copy(u_v, out_hbm.at[cc, pl.ds(nbase + k * _CHUNK, _CHUNK), :])

    @pl.when(ss == 0)
    def _():
        pltpu.sync_copy(cntacc, cnt_hbm.at[cc])


def _sc_edge(p, q, r, src, dst, ge, bne):
    mesh = plsc.VectorSubcoreMesh(
        core_axis_name="c", subcore_axis_name="s",
        num_cores=_NC, num_subcores=_NS,
    )
    fn = functools.partial(
        pl.kernel,
        compiler_params=pltpu.CompilerParams(needs_layout_passes=False),
        out_type=[
            jax.ShapeDtypeStruct((_NC, _NP, _D), jnp.float32),
            jax.ShapeDtypeStruct((_NC, _CR, _D), jnp.float32),
        ],
        mesh=mesh,
        scratch_types=[
            pltpu.VMEM_SHARED((_NP, _D), jnp.float32),
            pltpu.VMEM_SHARED((_CR, _D), jnp.float32),
            pltpu.VMEM((_CHUNK,), jnp.int32),
            pltpu.VMEM((_CHUNK,), jnp.int32),
            pltpu.VMEM((_CHUNK, _D), jnp.float32),
            pltpu.VMEM((_CHUNK, _D), jnp.float32),
            pltpu.VMEM((_CHUNK, _D), jnp.float32),
            pltpu.VMEM((_CR, _D), jnp.float32),
            pltpu.VMEM((_CHUNK,), jnp.int32),
            pltpu.VMEM((_D,), jnp.float32),
            pltpu.VMEM((_D,), jnp.float32),
            pltpu.SemaphoreType.DMA,
            pltpu.SemaphoreType.DMA,
        ],
    )(_sc_edge_body)
    return fn(p, q, r, src, dst, ge, bne)


def _node_body(part_ref, cnt_ref, h_ref, orig_ref, we2_ref, wa_ref, wb_ref,
               wc_ref, wn2_ref, be2_ref, bn1_ref, gn_ref, bnn_ref, bn2_ref,
               gf_ref, bf_ref, out_ref):
    asum = part_ref[0] + part_ref[1]
    cnt = cnt_ref[0] + cnt_ref[1]
    inv = 1.0 / jnp.maximum(cnt, 1.0)
    flag = jnp.minimum(cnt, 1.0)
    aggr = (
        jnp.dot(asum * inv, we2_ref[...], preferred_element_type=jnp.float32)
        + flag * be2_ref[...]
    )
    h = h_ref[...]
    x = (
        jnp.dot(h, wa_ref[...], preferred_element_type=jnp.float32)
        + jnp.dot(aggr, wb_ref[...], preferred_element_type=jnp.float32)
        + jnp.dot(orig_ref[...], wc_ref[...], preferred_element_type=jnp.float32)
        + bn1_ref[...]
    )
    x = jnp.maximum(x, 0.0)
    x = _ln_tc(x, gn_ref[...], bnn_ref[...])
    upd = jnp.dot(x, wn2_ref[...], preferred_element_type=jnp.float32) + bn2_ref[...]
    out = _SKIPW * upd + (1.0 - _SKIPW) * h
    out_ref[...] = _ln_tc(out, gf_ref[...], bf_ref[...])


def _node(part, cnt3, h, orig, we2, wn1a, wn1b, wn1c, wn2, rows):
    nb = 1000
    mat = lambda: pl.BlockSpec((_D, _D), lambda i: (0, 0))
    vec = lambda: pl.BlockSpec((1, _D), lambda i: (0, 0))
    return pl.pallas_call(
        _node_body,
        grid=(_N // nb,),
        in_specs=[
            pl.BlockSpec((_NC, nb, _D), lambda i: (0, i, 0)),
            pl.BlockSpec((_NC, nb, 1), lambda i: (0, i, 0)),
            pl.BlockSpec((nb, _D), lambda i: (i, 0)),
            pl.BlockSpec((nb, _D), lambda i: (i, 0)),
            mat(), mat(), mat(), mat(), mat(),
            vec(), vec(), vec(), vec(), vec(), vec(), vec(),
        ],
        out_specs=pl.BlockSpec((nb, _D), lambda i: (i, 0)),
        out_shape=jax.ShapeDtypeStruct((_N, _D), jnp.float32),
    )(part, cnt3, h, orig, we2, wn1a, wn1b, wn1c, wn2, *rows)


def kernel(coords_rec, h_feats_rec, original_receptor_node_features, edge_feat,
           We1, be1, ge, bne, We2, be2, Wn1, bn1, gn, bnn, Wn2, bn2, gf, bf,
           edge_index):
    del coords_rec
    src = edge_index[0].astype(jnp.int32)
    dst = edge_index[1].astype(jnp.int32)
    w1a = We1[:_D]
    w1b = We1[_D:2 * _D]
    w1c = We1[2 * _D:]

    p, q = _pq(h_feats_rec, w1a, w1b)
    r = _redge(edge_feat, w1c, be1.reshape(1, _D))
    part, cnt = _sc_edge(p, q, r, src, dst, ge, bne)
    cnt3 = cnt.reshape(_NC, _NP, 1)

    rows = [v.reshape(1, _D) for v in (be2, bn1, gn, bnn, bn2, gf, bf)]
    return _node(part, cnt3, h_feats_rec, original_receptor_node_features,
                 We2, Wn1[:_D], Wn1[_D:2 * _D], Wn1[2 * _D:], Wn2, rows)


# speedup vs baseline: 3.1067x; 1.2192x over previous
import functools

import jax
import jax.numpy as jnp
from jax import lax
from jax.experimental import pallas as pl
from jax.experimental.pallas import tpu as pltpu
from jax.experimental.pallas import tpu_sc as plsc

_N = 10000
_E = 320000
_D = 128
_DE = 16
_SKIPW = 0.5
_EPS = 1e-5

_NC = 2
_NS = 16
_EW = _E // (_NC * _NS)
_CHUNK = 80
_NCH = _EW // _CHUNK
_NP = 10240
_RPT = _NP // _NS
_RCH = _RPT // _CHUNK
_CR = _NP // _D


def _ln_tc(x, g, b):
    mu = jnp.mean(x, axis=-1, keepdims=True)
    var = jnp.mean((x - mu) ** 2, axis=-1, keepdims=True)
    return (x - mu) * lax.rsqrt(var + _EPS) * g + b


def _pq_body(h_ref, wa_ref, wb_ref, p_ref, q_ref):
    h = h_ref[...]
    p_ref[...] = jnp.dot(h, wa_ref[...], preferred_element_type=jnp.float32)
    q_ref[...] = jnp.dot(h, wb_ref[...], preferred_element_type=jnp.float32)


def _pq(h, wa, wb):
    nb = 1000
    return pl.pallas_call(
        _pq_body,
        grid=(_N // nb,),
        in_specs=[
            pl.BlockSpec((nb, _D), lambda i: (i, 0)),
            pl.BlockSpec((_D, _D), lambda i: (0, 0)),
            pl.BlockSpec((_D, _D), lambda i: (0, 0)),
        ],
        out_specs=[
            pl.BlockSpec((nb, _D), lambda i: (i, 0)),
            pl.BlockSpec((nb, _D), lambda i: (i, 0)),
        ],
        out_shape=[
            jax.ShapeDtypeStruct((_N, _D), jnp.float32),
            jax.ShapeDtypeStruct((_N, _D), jnp.float32),
        ],
    )(h, wa, wb)


def _r_body(ef_ref, wc_ref, be1_ref, r_ref):
    r_ref[...] = (
        jnp.dot(ef_ref[...], wc_ref[...], preferred_element_type=jnp.float32)
        + be1_ref[...]
    )


def _redge(ef, wc, be1_row):
    eb = 8000
    return pl.pallas_call(
        _r_body,
        grid=(_E // eb,),
        in_specs=[
            pl.BlockSpec((eb, _DE), lambda i: (i, 0)),
            pl.BlockSpec((_DE, _D), lambda i: (0, 0)),
            pl.BlockSpec((1, _D), lambda i: (0, 0)),
        ],
        out_specs=pl.BlockSpec((eb, _D), lambda i: (i, 0)),
        out_shape=jax.ShapeDtypeStruct((_E, _D), jnp.float32),
    )(ef, wc, be1_row)


def _sc_edge_body(p_hbm, q_hbm, r_hbm, src_hbm, dst_hbm,
                  out_hbm, cnt_hbm, acc, cntacc, src_v, dst_v,
                  r_v, u_v, cnt_v, rowidx_v, sem1, sem2):
    cc = lax.axis_index("c")
    ss = lax.axis_index("s")
    zero16 = jnp.zeros((16,), jnp.float32)
    one16 = jnp.full((16,), 1.0, jnp.float32)
    lane = lax.iota(jnp.int32, 16)
    cols = [jnp.full((16,), 16 * j, jnp.int32) + lane for j in range(_D // 16)]

    def _zrow(i, carry):
        rows = jnp.full((16,), i, jnp.int32)
        for j in range(_D // 16):
            plsc.store_scatter(u_v, [rows, cols[j]], zero16)
        return carry

    lax.fori_loop(0, _CHUNK, _zrow, 0)

    def _zcnt(i, carry):
        rows = jnp.full((16,), i, jnp.int32)
        for j in range(_D // 16):
            plsc.store_scatter(cnt_v, [rows, cols[j]], zero16)
        return carry

    lax.fori_loop(0, _CR, _zcnt, 0)
    for k in range(_CR // 16):
        rowidx_v[pl.ds(16 * k, 16)] = jnp.full((16,), 16 * k, jnp.int32) + lane

    nbase = ss * _RPT
    for k in range(_RCH):
        pltpu.sync_copy(u_v, acc.at[pl.ds(nbase + k * _CHUNK, _CHUNK), :])

    @pl.when(ss == 0)
    def _():
        for k in range(_CR // _CHUNK):
            pltpu.sync_copy(u_v, cntacc.at[pl.ds(k * _CHUNK, _CHUNK), :])

    plsc.subcore_barrier()

    ebase = (cc * _NS + ss) * _EW

    def _chunk(ci, carry):
        base = ebase + ci * _CHUNK
        cpa = pltpu.async_copy(src_hbm.at[pl.ds(base, _CHUNK)], src_v, sem1)
        cpb = pltpu.async_copy(dst_hbm.at[pl.ds(base, _CHUNK)], dst_v, sem2)
        cpa.wait()
        cpb.wait()
        pltpu.sync_copy(r_hbm.at[pl.ds(base, _CHUNK), :], r_v)
        pltpu.async_copy(p_hbm.at[src_v], r_v, sem1, add=True).wait()
        pltpu.async_copy(q_hbm.at[dst_v], r_v, sem2, add=True).wait()

        for k in range(_CHUNK // 16):
            idx = dst_v[pl.ds(16 * k, 16)]
            plsc.addupdate_scatter(cnt_v, [idx >> 7, idx & 127], one16)

        @plsc.parallel_loop(0, _CHUNK, 1, unroll=4)
        def _edge(e):
            rows = jnp.full((16,), e, jnp.int32)
            tot = zero16
            sq = zero16
            for j in range(_D // 16):
                t = jnp.maximum(plsc.load_gather(r_v, [rows, cols[j]]), zero16)
                tot = tot + t
                sq = sq + t * t
            def _allsum(x):
                return (jnp.cumsum(x) + jnp.flip(jnp.cumsum(jnp.flip(x, 0)), 0)
                        - x)

            mv = _allsum(tot) * (1.0 / _D)
            ex2 = _allsum(sq) * (1.0 / _D)
            xv = jnp.maximum(ex2 - mv * mv, zero16) + _EPS
            xi = plsc.bitcast(xv, jnp.int32)
            y = plsc.bitcast(jnp.full((16,), 0x5F3759DF, jnp.int32) - (xi >> 1),
                             jnp.float32)
            c15 = jnp.full((16,), 1.5, jnp.float32)
            ch = jnp.full((16,), 0.5, jnp.float32)
            for _ in range(3):
                y = y * (c15 - ch * xv * y * y)
            for j in range(_D // 16):
                t = jnp.maximum(plsc.load_gather(r_v, [rows, cols[j]]), zero16)
                plsc.store_scatter(u_v, [rows, cols[j]], (t - mv) * y)

        pltpu.sync_copy(u_v, acc.at[dst_v], add=True)
        return carry

    lax.fori_loop(0, _NCH, _chunk, 0)

    pltpu.sync_copy(cnt_v, cntacc.at[rowidx_v], add=True)
    plsc.subcore_barrier()

    for k in range(_RCH):
        pltpu.sync_copy(acc.at[pl.ds(nbase + k * _CHUNK, _CHUNK), :], u_v)
        pltpu.sync_copy(u_v, out_hbm.at[cc, pl.ds(nbase + k * _CHUNK, _CHUNK), :])

    @pl.when(ss == 0)
    def _():
        pltpu.sync_copy(cntacc, cnt_hbm.at[cc])


def _sc_edge(p, q, r, src, dst):
    mesh = plsc.VectorSubcoreMesh(
        core_axis_name="c", subcore_axis_name="s",
        num_cores=_NC, num_subcores=_NS,
    )
    fn = functools.partial(
        pl.kernel,
        compiler_params=pltpu.CompilerParams(needs_layout_passes=False),
        out_type=[
            jax.ShapeDtypeStruct((_NC, _NP, _D), jnp.float32),
            jax.ShapeDtypeStruct((_NC, _CR, _D), jnp.float32),
        ],
        mesh=mesh,
        scratch_types=[
            pltpu.VMEM_SHARED((_NP, _D), jnp.float32),
            pltpu.VMEM_SHARED((_CR, _D), jnp.float32),
            pltpu.VMEM((_CHUNK,), jnp.int32),
            pltpu.VMEM((_CHUNK,), jnp.int32),
            pltpu.VMEM((_CHUNK, _D), jnp.float32),
            pltpu.VMEM((_CHUNK, _D), jnp.float32),
            pltpu.VMEM((_CR, _D), jnp.float32),
            pltpu.VMEM((_CR,), jnp.int32),
            pltpu.SemaphoreType.DMA,
            pltpu.SemaphoreType.DMA,
        ],
    )(_sc_edge_body)
    return fn(p, q, r, src, dst)


def _node_body(part_ref, cnt_ref, h_ref, orig_ref, we2_ref, wa_ref, wb_ref,
               wc_ref, wn2_ref, ge_ref, bne_ref, be2_ref, bn1_ref, gn_ref,
               bnn_ref, bn2_ref, gf_ref, bf_ref, out_ref):
    cnt = cnt_ref[0] + cnt_ref[1]
    asum = (part_ref[0] + part_ref[1]) * ge_ref[...] + cnt * bne_ref[...]
    inv = 1.0 / jnp.maximum(cnt, 1.0)
    flag = jnp.minimum(cnt, 1.0)
    aggr = (
        jnp.dot(asum * inv, we2_ref[...], preferred_element_type=jnp.float32)
        + flag * be2_ref[...]
    )
    h = h_ref[...]
    x = (
        jnp.dot(h, wa_ref[...], preferred_element_type=jnp.float32)
        + jnp.dot(aggr, wb_ref[...], preferred_element_type=jnp.float32)
        + jnp.dot(orig_ref[...], wc_ref[...], preferred_element_type=jnp.float32)
        + bn1_ref[...]
    )
    x = jnp.maximum(x, 0.0)
    x = _ln_tc(x, gn_ref[...], bnn_ref[...])
    upd = jnp.dot(x, wn2_ref[...], preferred_element_type=jnp.float32) + bn2_ref[...]
    out = _SKIPW * upd + (1.0 - _SKIPW) * h
    out_ref[...] = _ln_tc(out, gf_ref[...], bf_ref[...])


def _node(part, cnt3, h, orig, we2, wn1a, wn1b, wn1c, wn2, rows):
    nb = 1000
    mat = lambda: pl.BlockSpec((_D, _D), lambda i: (0, 0))
    vec = lambda: pl.BlockSpec((1, _D), lambda i: (0, 0))
    return pl.pallas_call(
        _node_body,
        grid=(_N // nb,),
        in_specs=[
            pl.BlockSpec((_NC, nb, _D), lambda i: (0, i, 0)),
            pl.BlockSpec((_NC, nb, 1), lambda i: (0, i, 0)),
            pl.BlockSpec((nb, _D), lambda i: (i, 0)),
            pl.BlockSpec((nb, _D), lambda i: (i, 0)),
            mat(), mat(), mat(), mat(), mat(),
            vec(), vec(), vec(), vec(), vec(), vec(), vec(), vec(), vec(),
        ],
        out_specs=pl.BlockSpec((nb, _D), lambda i: (i, 0)),
        out_shape=jax.ShapeDtypeStruct((_N, _D), jnp.float32),
    )(part, cnt3, h, orig, we2, wn1a, wn1b, wn1c, wn2, *rows)


def kernel(coords_rec, h_feats_rec, original_receptor_node_features, edge_feat,
           We1, be1, ge, bne, We2, be2, Wn1, bn1, gn, bnn, Wn2, bn2, gf, bf,
           edge_index):
    del coords_rec
    src = edge_index[0].astype(jnp.int32)
    dst = edge_index[1].astype(jnp.int32)
    w1a = We1[:_D]
    w1b = We1[_D:2 * _D]
    w1c = We1[2 * _D:]

    p, q = _pq(h_feats_rec, w1a, w1b)
    r = _redge(edge_feat, w1c, be1.reshape(1, _D))
    part, cnt = _sc_edge(p, q, r, src, dst)
    cnt3 = cnt.reshape(_NC, _NP, 1)

    rows = [v.reshape(1, _D)
            for v in (ge, bne, be2, bn1, gn, bnn, bn2, gf, bf)]
    return _node(part, cnt3, h_feats_rec, original_receptor_node_features,
                 We2, Wn1[:_D], Wn1[_D:2 * _D], Wn1[2 * _D:], Wn2, rows)


# speedup vs baseline: 3.6168x; 1.1642x over previous
import functools

import jax
import jax.numpy as jnp
from jax import lax
from jax.experimental import pallas as pl
from jax.experimental.pallas import tpu as pltpu
from jax.experimental.pallas import tpu_sc as plsc

_N = 10000
_E = 320000
_D = 128
_DE = 16
_SKIPW = 0.5
_EPS = 1e-5

_NC = 2
_NS = 16
_EW = _E // (_NC * _NS)
_CHUNK = 80
_NCH = _EW // _CHUNK
_NP = 10240
_RPT = _NP // _NS
_RCH = _RPT // _CHUNK
_CR = _NP // _D


def _ln_tc(x, g, b):
    mu = jnp.mean(x, axis=-1, keepdims=True)
    var = jnp.mean((x - mu) ** 2, axis=-1, keepdims=True)
    return (x - mu) * lax.rsqrt(var + _EPS) * g + b


def _pq_body(h_ref, wa_ref, wb_ref, p_ref, q_ref):
    h = h_ref[...]
    p_ref[...] = jnp.dot(h, wa_ref[...], preferred_element_type=jnp.float32)
    q_ref[...] = jnp.dot(h, wb_ref[...], preferred_element_type=jnp.float32)


def _pq(h, wa, wb):
    nb = 1000
    return pl.pallas_call(
        _pq_body,
        grid=(_N // nb,),
        in_specs=[
            pl.BlockSpec((nb, _D), lambda i: (i, 0)),
            pl.BlockSpec((_D, _D), lambda i: (0, 0)),
            pl.BlockSpec((_D, _D), lambda i: (0, 0)),
        ],
        out_specs=[
            pl.BlockSpec((nb, _D), lambda i: (i, 0)),
            pl.BlockSpec((nb, _D), lambda i: (i, 0)),
        ],
        out_shape=[
            jax.ShapeDtypeStruct((_N, _D), jnp.float32),
            jax.ShapeDtypeStruct((_N, _D), jnp.float32),
        ],
    )(h, wa, wb)


def _r_body(ef_ref, wc_ref, be1_ref, r_ref):
    r_ref[...] = (
        jnp.dot(ef_ref[...], wc_ref[...], preferred_element_type=jnp.float32)
        + be1_ref[...]
    )


def _redge(ef, wc, be1_row):
    eb = 8000
    return pl.pallas_call(
        _r_body,
        grid=(_E // eb,),
        in_specs=[
            pl.BlockSpec((eb, _DE), lambda i: (i, 0)),
            pl.BlockSpec((_DE, _D), lambda i: (0, 0)),
            pl.BlockSpec((1, _D), lambda i: (0, 0)),
        ],
        out_specs=pl.BlockSpec((eb, _D), lambda i: (i, 0)),
        out_shape=jax.ShapeDtypeStruct((_E, _D), jnp.float32),
    )(ef, wc, be1_row)


def _sc_edge_body(p_hbm, q_hbm, r_hbm, src_hbm, dst_hbm,
                  out_hbm, cnt_hbm, acc, cntacc, src_v, dst_v,
                  r_v, u_v, cnt_v, rowidx_v, sem1, sem2):
    cc = lax.axis_index("c")
    ss = lax.axis_index("s")
    zero16 = jnp.zeros((16,), jnp.float32)
    one16 = jnp.full((16,), 1.0, jnp.float32)
    lane = lax.iota(jnp.int32, 16)
    cols = [jnp.full((16,), 16 * j, jnp.int32) + lane for j in range(_D // 16)]

    def _zrow(i, carry):
        rows = jnp.full((16,), i, jnp.int32)
        for j in range(_D // 16):
            plsc.store_scatter(u_v, [rows, cols[j]], zero16)
        return carry

    lax.fori_loop(0, _CHUNK, _zrow, 0)

    def _zcnt(i, carry):
        rows = jnp.full((16,), i, jnp.int32)
        for j in range(_D // 16):
            plsc.store_scatter(cnt_v, [rows, cols[j]], zero16)
        return carry

    lax.fori_loop(0, _CR, _zcnt, 0)
    for k in range(_CR // 16):
        rowidx_v[pl.ds(16 * k, 16)] = jnp.full((16,), 16 * k, jnp.int32) + lane

    nbase = ss * _RPT
    for k in range(_RCH):
        pltpu.sync_copy(u_v, acc.at[pl.ds(nbase + k * _CHUNK, _CHUNK), :])

    @pl.when(ss == 0)
    def _():
        for k in range(_CR // _CHUNK):
            pltpu.sync_copy(u_v, cntacc.at[pl.ds(k * _CHUNK, _CHUNK), :])

    plsc.subcore_barrier()

    ebase = (cc * _NS + ss) * _EW

    def _chunk(ci, carry):
        base = ebase + ci * _CHUNK
        cpa = pltpu.async_copy(src_hbm.at[pl.ds(base, _CHUNK)], src_v, sem1)
        cpb = pltpu.async_copy(dst_hbm.at[pl.ds(base, _CHUNK)], dst_v, sem2)
        cpa.wait()
        cpb.wait()
        pltpu.sync_copy(r_hbm.at[pl.ds(base, _CHUNK), :], r_v)
        pltpu.async_copy(p_hbm.at[src_v], r_v, sem1, add=True).wait()
        pltpu.async_copy(q_hbm.at[dst_v], r_v, sem2, add=True).wait()

        for k in range(_CHUNK // 16):
            idx = dst_v[pl.ds(16 * k, 16)]
            plsc.addupdate_scatter(cnt_v, [idx >> 7, idx & 127], one16)

        @plsc.parallel_loop(0, _CHUNK, 1, unroll=2)
        def _edge(e):
            tot = zero16
            sq = zero16
            for j in range(_D // 16):
                t = jnp.maximum(r_v[e, pl.ds(16 * j, 16)], zero16)
                tot = tot + t
                sq = sq + t * t

            def _allsum(x):
                return (jnp.cumsum(x) + jnp.flip(jnp.cumsum(jnp.flip(x, 0)), 0)
                        - x)

            mv = _allsum(tot) * (1.0 / _D)
            ex2 = _allsum(sq) * (1.0 / _D)
            xv = jnp.maximum(ex2 - mv * mv, zero16) + _EPS
            xi = plsc.bitcast(xv, jnp.int32)
            y = plsc.bitcast(jnp.full((16,), 0x5F3759DF, jnp.int32) - (xi >> 1),
                             jnp.float32)
            c15 = jnp.full((16,), 1.5, jnp.float32)
            ch = jnp.full((16,), 0.5, jnp.float32)
            for _ in range(3):
                y = y * (c15 - ch * xv * y * y)
            for j in range(_D // 16):
                t = jnp.maximum(r_v[e, pl.ds(16 * j, 16)], zero16)
                u_v[e, pl.ds(16 * j, 16)] = (t - mv) * y

        pltpu.sync_copy(u_v, acc.at[dst_v], add=True)
        return carry

    lax.fori_loop(0, _NCH, _chunk, 0)

    pltpu.sync_copy(cnt_v, cntacc.at[rowidx_v], add=True)
    plsc.subcore_barrier()

    for k in range(_RCH):
        pltpu.sync_copy(acc.at[pl.ds(nbase + k * _CHUNK, _CHUNK), :], u_v)
        pltpu.sync_copy(u_v, out_hbm.at[cc, pl.ds(nbase + k * _CHUNK, _CHUNK), :])

    @pl.when(ss == 0)
    def _():
        pltpu.sync_copy(cntacc, cnt_hbm.at[cc])


def _sc_edge(p, q, r, src, dst):
    mesh = plsc.VectorSubcoreMesh(
        core_axis_name="c", subcore_axis_name="s",
        num_cores=_NC, num_subcores=_NS,
    )
    fn = functools.partial(
        pl.kernel,
        compiler_params=pltpu.CompilerParams(needs_layout_passes=False, use_tc_tiling_on_sc=False),
        out_type=[
            jax.ShapeDtypeStruct((_NC, _NP, _D), jnp.float32),
            jax.ShapeDtypeStruct((_NC, _CR, _D), jnp.float32),
        ],
        mesh=mesh,
        scratch_types=[
            pltpu.VMEM_SHARED((_NP, _D), jnp.float32),
            pltpu.VMEM_SHARED((_CR, _D), jnp.float32),
            pltpu.VMEM((_CHUNK,), jnp.int32),
            pltpu.VMEM((_CHUNK,), jnp.int32),
            pltpu.VMEM((_CHUNK, _D), jnp.float32),
            pltpu.VMEM((_CHUNK, _D), jnp.float32),
            pltpu.VMEM((_CR, _D), jnp.float32),
            pltpu.VMEM((_CR,), jnp.int32),
            pltpu.SemaphoreType.DMA,
            pltpu.SemaphoreType.DMA,
        ],
    )(_sc_edge_body)
    return fn(p, q, r, src, dst)


def _node_body(part_ref, cnt_ref, h_ref, orig_ref, we2_ref, wa_ref, wb_ref,
               wc_ref, wn2_ref, ge_ref, bne_ref, be2_ref, bn1_ref, gn_ref,
               bnn_ref, bn2_ref, gf_ref, bf_ref, out_ref):
    cnt = cnt_ref[0] + cnt_ref[1]
    asum = (part_ref[0] + part_ref[1]) * ge_ref[...] + cnt * bne_ref[...]
    inv = 1.0 / jnp.maximum(cnt, 1.0)
    flag = jnp.minimum(cnt, 1.0)
    aggr = (
        jnp.dot(asum * inv, we2_ref[...], preferred_element_type=jnp.float32)
        + flag * be2_ref[...]
    )
    h = h_ref[...]
    x = (
        jnp.dot(h, wa_ref[...], preferred_element_type=jnp.float32)
        + jnp.dot(aggr, wb_ref[...], preferred_element_type=jnp.float32)
        + jnp.dot(orig_ref[...], wc_ref[...], preferred_element_type=jnp.float32)
        + bn1_ref[...]
    )
    x = jnp.maximum(x, 0.0)
    x = _ln_tc(x, gn_ref[...], bnn_ref[...])
    upd = jnp.dot(x, wn2_ref[...], preferred_element_type=jnp.float32) + bn2_ref[...]
    out = _SKIPW * upd + (1.0 - _SKIPW) * h
    out_ref[...] = _ln_tc(out, gf_ref[...], bf_ref[...])


def _node(part, cnt3, h, orig, we2, wn1a, wn1b, wn1c, wn2, rows):
    nb = 1000
    mat = lambda: pl.BlockSpec((_D, _D), lambda i: (0, 0))
    vec = lambda: pl.BlockSpec((1, _D), lambda i: (0, 0))
    return pl.pallas_call(
        _node_body,
        grid=(_N // nb,),
        in_specs=[
            pl.BlockSpec((_NC, nb, _D), lambda i: (0, i, 0)),
            pl.BlockSpec((_NC, nb, 1), lambda i: (0, i, 0)),
            pl.BlockSpec((nb, _D), lambda i: (i, 0)),
            pl.BlockSpec((nb, _D), lambda i: (i, 0)),
            mat(), mat(), mat(), mat(), mat(),
            vec(), vec(), vec(), vec(), vec(), vec(), vec(), vec(), vec(),
        ],
        out_specs=pl.BlockSpec((nb, _D), lambda i: (i, 0)),
        out_shape=jax.ShapeDtypeStruct((_N, _D), jnp.float32),
    )(part, cnt3, h, orig, we2, wn1a, wn1b, wn1c, wn2, *rows)


def kernel(coords_rec, h_feats_rec, original_receptor_node_features, edge_feat,
           We1, be1, ge, bne, We2, be2, Wn1, bn1, gn, bnn, Wn2, bn2, gf, bf,
           edge_index):
    del coords_rec
    src = edge_index[0].astype(jnp.int32)
    dst = edge_index[1].astype(jnp.int32)
    w1a = We1[:_D]
    w1b = We1[_D:2 * _D]
    w1c = We1[2 * _D:]

    p, q = _pq(h_feats_rec, w1a, w1b)
    r = _redge(edge_feat, w1c, be1.reshape(1, _D))
    part, cnt = _sc_edge(p, q, r, src, dst)
    cnt3 = cnt.reshape(_NC, _NP, 1)

    rows = [v.reshape(1, _D)
            for v in (ge, bne, be2, bn1, gn, bnn, bn2, gf, bf)]
    return _node(part, cnt3, h_feats_rec, original_receptor_node_features,
                 We2, Wn1[:_D], Wn1[_D:2 * _D], Wn1[2 * _D:], Wn2, rows)


# speedup vs baseline: 3.6605x; 1.0121x over previous
import functools

import jax
import jax.numpy as jnp
from jax import lax
from jax.experimental import pallas as pl
from jax.experimental.pallas import tpu as pltpu
from jax.experimental.pallas import tpu_sc as plsc

_N = 10000
_E = 320000
_D = 128
_DE = 16
_SKIPW = 0.5
_EPS = 1e-5

_NC = 2
_NS = 16
_EW = _E // (_NC * _NS)
_CHUNK = 80
_NCH = _EW // _CHUNK
_NP = 10240
_RPT = _NP // _NS
_RCH = _RPT // _CHUNK
_CR = _NP // _D


def _ln_tc(x, g, b):
    mu = jnp.mean(x, axis=-1, keepdims=True)
    var = jnp.mean((x - mu) ** 2, axis=-1, keepdims=True)
    return (x - mu) * lax.rsqrt(var + _EPS) * g + b


def _pq_body(h_ref, wa_ref, wb_ref, p_ref, q_ref):
    h = h_ref[...]
    p_ref[...] = jnp.dot(h, wa_ref[...], preferred_element_type=jnp.float32)
    q_ref[...] = jnp.dot(h, wb_ref[...], preferred_element_type=jnp.float32)


def _pq(h, wa, wb):
    nb = 1000
    return pl.pallas_call(
        _pq_body,
        grid=(_N // nb,),
        in_specs=[
            pl.BlockSpec((nb, _D), lambda i: (i, 0)),
            pl.BlockSpec((_D, _D), lambda i: (0, 0)),
            pl.BlockSpec((_D, _D), lambda i: (0, 0)),
        ],
        out_specs=[
            pl.BlockSpec((nb, _D), lambda i: (i, 0)),
            pl.BlockSpec((nb, _D), lambda i: (i, 0)),
        ],
        out_shape=[
            jax.ShapeDtypeStruct((_N, _D), jnp.float32),
            jax.ShapeDtypeStruct((_N, _D), jnp.float32),
        ],
    )(h, wa, wb)


def _r_body(ef_ref, wc_ref, be1_ref, r_ref):
    r_ref[...] = (
        jnp.dot(ef_ref[...], wc_ref[...], preferred_element_type=jnp.float32)
        + be1_ref[...]
    )


def _redge(ef, wc, be1_row):
    eb = 8000
    return pl.pallas_call(
        _r_body,
        grid=(_E // eb,),
        in_specs=[
            pl.BlockSpec((eb, _DE), lambda i: (i, 0)),
            pl.BlockSpec((_DE, _D), lambda i: (0, 0)),
            pl.BlockSpec((1, _D), lambda i: (0, 0)),
        ],
        out_specs=pl.BlockSpec((eb, _D), lambda i: (i, 0)),
        out_shape=jax.ShapeDtypeStruct((_E, _D), jnp.float32),
    )(ef, wc, be1_row)


def _sc_edge_body(p_hbm, q_hbm, r_hbm, src_hbm, dst_hbm,
                  out_hbm, cnt_hbm, acc, cntacc, src_v, dst_v,
                  r_v, u_v, cnt_v, rowidx_v, st_v, sq_v, mv_v, y_v,
                  sem1, sem2):
    cc = lax.axis_index("c")
    ss = lax.axis_index("s")
    zero16 = jnp.zeros((16,), jnp.float32)
    one16 = jnp.full((16,), 1.0, jnp.float32)
    lane = lax.iota(jnp.int32, 16)
    cols = [jnp.full((16,), 16 * j, jnp.int32) + lane for j in range(_D // 16)]

    def _zrow(i, carry):
        rows = jnp.full((16,), i, jnp.int32)
        for j in range(_D // 16):
            plsc.store_scatter(u_v, [rows, cols[j]], zero16)
        return carry

    lax.fori_loop(0, _CHUNK, _zrow, 0)

    def _zcnt(i, carry):
        rows = jnp.full((16,), i, jnp.int32)
        for j in range(_D // 16):
            plsc.store_scatter(cnt_v, [rows, cols[j]], zero16)
        return carry

    lax.fori_loop(0, _CR, _zcnt, 0)
    for k in range(_CR // 16):
        rowidx_v[pl.ds(16 * k, 16)] = jnp.full((16,), 16 * k, jnp.int32) + lane

    nbase = ss * _RPT
    for k in range(_RCH):
        pltpu.sync_copy(u_v, acc.at[pl.ds(nbase + k * _CHUNK, _CHUNK), :])

    @pl.when(ss == 0)
    def _():
        for k in range(_CR // _CHUNK):
            pltpu.sync_copy(u_v, cntacc.at[pl.ds(k * _CHUNK, _CHUNK), :])

    plsc.subcore_barrier()

    ebase = (cc * _NS + ss) * _EW

    def _chunk(ci, carry):
        base = ebase + ci * _CHUNK
        cpa = pltpu.async_copy(src_hbm.at[pl.ds(base, _CHUNK)], src_v, sem1)
        cpb = pltpu.async_copy(dst_hbm.at[pl.ds(base, _CHUNK)], dst_v, sem2)
        cpa.wait()
        cpb.wait()
        pltpu.sync_copy(r_hbm.at[pl.ds(base, _CHUNK), :], r_v)
        pltpu.async_copy(p_hbm.at[src_v], r_v, sem1, add=True).wait()
        pltpu.async_copy(q_hbm.at[dst_v], r_v, sem2, add=True).wait()

        for k in range(_CHUNK // 16):
            idx = dst_v[pl.ds(16 * k, 16)]
            plsc.addupdate_scatter(cnt_v, [idx >> 7, idx & 127], one16)

        @plsc.parallel_loop(0, _CHUNK, 1, unroll=2)
        def _pA(e):
            tot = zero16
            sq = zero16
            for j in range(_D // 16):
                t = jnp.maximum(r_v[e, pl.ds(16 * j, 16)], zero16)
                tot = tot + t
                sq = sq + t * t
            st_v[pl.ds(e * 17, 16)] = tot
            sq_v[pl.ds(e * 17, 16)] = sq

        c15 = jnp.full((16,), 1.5, jnp.float32)
        ch = jnp.full((16,), 0.5, jnp.float32)
        for g in range(_CHUNK // 16):
            gidx = (lane + jnp.full((16,), 16 * g, jnp.int32)) * 17
            tot16 = zero16
            sq16 = zero16
            for l in range(16):
                il = gidx + jnp.full((16,), l, jnp.int32)
                tot16 = tot16 + plsc.load_gather(st_v, [il])
                sq16 = sq16 + plsc.load_gather(sq_v, [il])
            m16 = tot16 * (1.0 / _D)
            xv = jnp.maximum(sq16 * (1.0 / _D) - m16 * m16, zero16) + _EPS
            xi = plsc.bitcast(xv, jnp.int32)
            y = plsc.bitcast(jnp.full((16,), 0x5F3759DF, jnp.int32) - (xi >> 1),
                             jnp.float32)
            for _ in range(3):
                y = y * (c15 - ch * xv * y * y)
            mv_v[pl.ds(16 * g, 16)] = m16
            y_v[pl.ds(16 * g, 16)] = y

        @plsc.parallel_loop(0, _CHUNK, 1, unroll=2)
        def _pC(e):
            eb16 = jnp.full((16,), e, jnp.int32)
            mvb = plsc.load_gather(mv_v, [eb16])
            yb = plsc.load_gather(y_v, [eb16])
            for j in range(_D // 16):
                t = jnp.maximum(r_v[e, pl.ds(16 * j, 16)], zero16)
                u_v[e, pl.ds(16 * j, 16)] = (t - mvb) * yb

        pltpu.sync_copy(u_v, acc.at[dst_v], add=True)
        return carry

    lax.fori_loop(0, _NCH, _chunk, 0)

    pltpu.sync_copy(cnt_v, cntacc.at[rowidx_v], add=True)
    plsc.subcore_barrier()

    for k in range(_RCH):
        pltpu.sync_copy(acc.at[pl.ds(nbase + k * _CHUNK, _CHUNK), :], u_v)
        pltpu.sync_copy(u_v, out_hbm.at[cc, pl.ds(nbase + k * _CHUNK, _CHUNK), :])

    @pl.when(ss == 0)
    def _():
        pltpu.sync_copy(cntacc, cnt_hbm.at[cc])


def _sc_edge(p, q, r, src, dst):
    mesh = plsc.VectorSubcoreMesh(
        core_axis_name="c", subcore_axis_name="s",
        num_cores=_NC, num_subcores=_NS,
    )
    fn = functools.partial(
        pl.kernel,
        compiler_params=pltpu.CompilerParams(needs_layout_passes=False, use_tc_tiling_on_sc=False),
        out_type=[
            jax.ShapeDtypeStruct((_NC, _NP, _D), jnp.float32),
            jax.ShapeDtypeStruct((_NC, _CR, _D), jnp.float32),
        ],
        mesh=mesh,
        scratch_types=[
            pltpu.VMEM_SHARED((_NP, _D), jnp.float32),
            pltpu.VMEM_SHARED((_CR, _D), jnp.float32),
            pltpu.VMEM((_CHUNK,), jnp.int32),
            pltpu.VMEM((_CHUNK,), jnp.int32),
            pltpu.VMEM((_CHUNK, _D), jnp.float32),
            pltpu.VMEM((_CHUNK, _D), jnp.float32),
            pltpu.VMEM((_CR, _D), jnp.float32),
            pltpu.VMEM((_CR,), jnp.int32),
            pltpu.VMEM((_CHUNK * 17 + 16,), jnp.float32),
            pltpu.VMEM((_CHUNK * 17 + 16,), jnp.float32),
            pltpu.VMEM((_CHUNK,), jnp.float32),
            pltpu.VMEM((_CHUNK,), jnp.float32),
            pltpu.SemaphoreType.DMA,
            pltpu.SemaphoreType.DMA,
        ],
    )(_sc_edge_body)
    return fn(p, q, r, src, dst)


def _node_body(part_ref, cnt_ref, h_ref, orig_ref, we2_ref, wa_ref, wb_ref,
               wc_ref, wn2_ref, ge_ref, bne_ref, be2_ref, bn1_ref, gn_ref,
               bnn_ref, bn2_ref, gf_ref, bf_ref, out_ref):
    cnt = cnt_ref[0] + cnt_ref[1]
    asum = (part_ref[0] + part_ref[1]) * ge_ref[...] + cnt * bne_ref[...]
    inv = 1.0 / jnp.maximum(cnt, 1.0)
    flag = jnp.minimum(cnt, 1.0)
    aggr = (
        jnp.dot(asum * inv, we2_ref[...], preferred_element_type=jnp.float32)
        + flag * be2_ref[...]
    )
    h = h_ref[...]
    x = (
        jnp.dot(h, wa_ref[...], preferred_element_type=jnp.float32)
        + jnp.dot(aggr, wb_ref[...], preferred_element_type=jnp.float32)
        + jnp.dot(orig_ref[...], wc_ref[...], preferred_element_type=jnp.float32)
        + bn1_ref[...]
    )
    x = jnp.maximum(x, 0.0)
    x = _ln_tc(x, gn_ref[...], bnn_ref[...])
    upd = jnp.dot(x, wn2_ref[...], preferred_element_type=jnp.float32) + bn2_ref[...]
    out = _SKIPW * upd + (1.0 - _SKIPW) * h
    out_ref[...] = _ln_tc(out, gf_ref[...], bf_ref[...])


def _node(part, cnt3, h, orig, we2, wn1a, wn1b, wn1c, wn2, rows):
    nb = 1000
    mat = lambda: pl.BlockSpec((_D, _D), lambda i: (0, 0))
    vec = lambda: pl.BlockSpec((1, _D), lambda i: (0, 0))
    return pl.pallas_call(
        _node_body,
        grid=(_N // nb,),
        in_specs=[
            pl.BlockSpec((_NC, nb, _D), lambda i: (0, i, 0)),
            pl.BlockSpec((_NC, nb, 1), lambda i: (0, i, 0)),
            pl.BlockSpec((nb, _D), lambda i: (i, 0)),
            pl.BlockSpec((nb, _D), lambda i: (i, 0)),
            mat(), mat(), mat(), mat(), mat(),
            vec(), vec(), vec(), vec(), vec(), vec(), vec(), vec(), vec(),
        ],
        out_specs=pl.BlockSpec((nb, _D), lambda i: (i, 0)),
        out_shape=jax.ShapeDtypeStruct((_N, _D), jnp.float32),
    )(part, cnt3, h, orig, we2, wn1a, wn1b, wn1c, wn2, *rows)


def kernel(coords_rec, h_feats_rec, original_receptor_node_features, edge_feat,
           We1, be1, ge, bne, We2, be2, Wn1, bn1, gn, bnn, Wn2, bn2, gf, bf,
           edge_index):
    del coords_rec
    src = edge_index[0].astype(jnp.int32)
    dst = edge_index[1].astype(jnp.int32)
    w1a = We1[:_D]
    w1b = We1[_D:2 * _D]
    w1c = We1[2 * _D:]

    p, q = _pq(h_feats_rec, w1a, w1b)
    r = _redge(edge_feat, w1c, be1.reshape(1, _D))
    part, cnt = _sc_edge(p, q, r, src, dst)
    cnt3 = cnt.reshape(_NC, _NP, 1)

    rows = [v.reshape(1, _D)
            for v in (ge, bne, be2, bn1, gn, bnn, bn2, gf, bf)]
    return _node(part, cnt3, h_feats_rec, original_receptor_node_features,
                 We2, Wn1[:_D], Wn1[_D:2 * _D], Wn1[2 * _D:], Wn2, rows)


# speedup vs baseline: 4.3774x; 1.1958x over previous
import functools

import jax
import jax.numpy as jnp
from jax import lax
from jax.experimental import pallas as pl
from jax.experimental.pallas import tpu as pltpu
from jax.experimental.pallas import tpu_sc as plsc

_N = 10000
_E = 320000
_D = 128
_DE = 16
_SKIPW = 0.5
_EPS = 1e-5

_NC = 2
_NS = 16
_EW = _E // (_NC * _NS)
_CHUNK = 80
_NCH = _EW // _CHUNK
_NP = 10240
_RPT = _NP // _NS
_RCH = _RPT // _CHUNK
_CR = _NP // _D


def _ln_tc(x, g, b):
    mu = jnp.mean(x, axis=-1, keepdims=True)
    var = jnp.mean((x - mu) ** 2, axis=-1, keepdims=True)
    return (x - mu) * lax.rsqrt(var + _EPS) * g + b


def _pq_body(h_ref, wa_ref, wb_ref, p_ref, q_ref):
    h = h_ref[...]
    p_ref[...] = jnp.dot(h, wa_ref[...], preferred_element_type=jnp.float32)
    q_ref[...] = jnp.dot(h, wb_ref[...], preferred_element_type=jnp.float32)


def _pq(h, wa, wb):
    nb = 1000
    return pl.pallas_call(
        _pq_body,
        grid=(_N // nb,),
        in_specs=[
            pl.BlockSpec((nb, _D), lambda i: (i, 0)),
            pl.BlockSpec((_D, _D), lambda i: (0, 0)),
            pl.BlockSpec((_D, _D), lambda i: (0, 0)),
        ],
        out_specs=[
            pl.BlockSpec((nb, _D), lambda i: (i, 0)),
            pl.BlockSpec((nb, _D), lambda i: (i, 0)),
        ],
        out_shape=[
            jax.ShapeDtypeStruct((_N, _D), jnp.float32),
            jax.ShapeDtypeStruct((_N, _D), jnp.float32),
        ],
    )(h, wa, wb)


def _r_body(ef_ref, wc_ref, be1_ref, r_ref):
    r_ref[...] = (
        jnp.dot(ef_ref[...], wc_ref[...], preferred_element_type=jnp.float32)
        + be1_ref[...]
    )


def _redge(ef, wc, be1_row):
    eb = 8000
    return pl.pallas_call(
        _r_body,
        grid=(_E // eb,),
        in_specs=[
            pl.BlockSpec((eb, _DE), lambda i: (i, 0)),
            pl.BlockSpec((_DE, _D), lambda i: (0, 0)),
            pl.BlockSpec((1, _D), lambda i: (0, 0)),
        ],
        out_specs=pl.BlockSpec((eb, _D), lambda i: (i, 0)),
        out_shape=jax.ShapeDtypeStruct((_E, _D), jnp.float32),
    )(ef, wc, be1_row)


def _sc_edge_body(p_hbm, q_hbm, r_hbm, src_hbm, dst_hbm,
                  out_hbm, cnt_hbm, acc, cntacc, src_v, dst_v,
                  r_v, u_v, cnt_v, rowidx_v, st_v, sq_v, mv_v, y_v,
                  sem1, sem2, sem3):
    cc = lax.axis_index("c")
    ss = lax.axis_index("s")
    zero16 = jnp.zeros((16,), jnp.float32)
    one16 = jnp.full((16,), 1.0, jnp.float32)
    lane = lax.iota(jnp.int32, 16)
    cols = [jnp.full((16,), 16 * j, jnp.int32) + lane for j in range(_D // 16)]

    def _zrow(i, carry):
        rows = jnp.full((16,), i, jnp.int32)
        for j in range(_D // 16):
            plsc.store_scatter(u_v, [rows, cols[j]], zero16)
        return carry

    lax.fori_loop(0, _CHUNK, _zrow, 0)

    def _zcnt(i, carry):
        rows = jnp.full((16,), i, jnp.int32)
        for j in range(_D // 16):
            plsc.store_scatter(cnt_v, [rows, cols[j]], zero16)
        return carry

    lax.fori_loop(0, _CR, _zcnt, 0)
    for k in range(_CR // 16):
        rowidx_v[pl.ds(16 * k, 16)] = jnp.full((16,), 16 * k, jnp.int32) + lane

    nbase = ss * _RPT
    for k in range(_RCH):
        pltpu.sync_copy(u_v, acc.at[pl.ds(nbase + k * _CHUNK, _CHUNK), :])

    @pl.when(ss == 0)
    def _():
        for k in range(_CR // _CHUNK):
            pltpu.sync_copy(u_v, cntacc.at[pl.ds(k * _CHUNK, _CHUNK), :])

    plsc.subcore_barrier()

    ebase = (cc * _NS + ss) * _EW

    def _chunk(ci, carry):
        base = ebase + ci * _CHUNK
        cpr = pltpu.async_copy(r_hbm.at[pl.ds(base, _CHUNK), :], r_v, sem3)
        cpa = pltpu.async_copy(src_hbm.at[pl.ds(base, _CHUNK)], src_v, sem1)
        cpb = pltpu.async_copy(dst_hbm.at[pl.ds(base, _CHUNK)], dst_v, sem2)
        cpa.wait()
        cpb.wait()
        cpr.wait()
        cp1 = pltpu.async_copy(p_hbm.at[src_v], r_v, sem1, add=True)
        cp2 = pltpu.async_copy(q_hbm.at[dst_v], r_v, sem2, add=True)
        cp1.wait()
        cp2.wait()

        for k in range(_CHUNK // 16):
            idx = dst_v[pl.ds(16 * k, 16)]
            plsc.addupdate_scatter(cnt_v, [idx >> 7, idx & 127], one16)

        @plsc.parallel_loop(0, _CHUNK, 1, unroll=2)
        def _pA(e):
            tot = zero16
            sq = zero16
            for j in range(_D // 16):
                t = jnp.maximum(r_v[e, pl.ds(16 * j, 16)], zero16)
                tot = tot + t
                sq = sq + t * t
            st_v[pl.ds(e * 17, 16)] = tot
            sq_v[pl.ds(e * 17, 16)] = sq

        c15 = jnp.full((16,), 1.5, jnp.float32)
        ch = jnp.full((16,), 0.5, jnp.float32)
        for g in range(_CHUNK // 16):
            gidx = (lane + jnp.full((16,), 16 * g, jnp.int32)) * 17
            tot16 = zero16
            sq16 = zero16
            for l in range(16):
                il = gidx + jnp.full((16,), l, jnp.int32)
                tot16 = tot16 + plsc.load_gather(st_v, [il])
                sq16 = sq16 + plsc.load_gather(sq_v, [il])
            m16 = tot16 * (1.0 / _D)
            xv = jnp.maximum(sq16 * (1.0 / _D) - m16 * m16, zero16) + _EPS
            xi = plsc.bitcast(xv, jnp.int32)
            y = plsc.bitcast(jnp.full((16,), 0x5F3759DF, jnp.int32) - (xi >> 1),
                             jnp.float32)
            for _ in range(3):
                y = y * (c15 - ch * xv * y * y)
            mv_v[pl.ds(16 * g, 16)] = m16
            y_v[pl.ds(16 * g, 16)] = y

        @plsc.parallel_loop(0, _CHUNK, 1, unroll=2)
        def _pC(e):
            eb16 = jnp.full((16,), e, jnp.int32)
            mvb = plsc.load_gather(mv_v, [eb16])
            yb = plsc.load_gather(y_v, [eb16])
            for j in range(_D // 16):
                t = jnp.maximum(r_v[e, pl.ds(16 * j, 16)], zero16)
                u_v[e, pl.ds(16 * j, 16)] = (t - mvb) * yb

        pltpu.sync_copy(u_v, acc.at[dst_v], add=True)
        return carry

    lax.fori_loop(0, _NCH, _chunk, 0)

    pltpu.sync_copy(cnt_v, cntacc.at[rowidx_v], add=True)
    plsc.subcore_barrier()

    for k in range(_RCH):
        pltpu.sync_copy(acc.at[pl.ds(nbase + k * _CHUNK, _CHUNK), :], u_v)
        pltpu.sync_copy(u_v, out_hbm.at[cc, pl.ds(nbase + k * _CHUNK, _CHUNK), :])

    @pl.when(ss == 0)
    def _():
        pltpu.sync_copy(cntacc, cnt_hbm.at[cc])


def _sc_edge(p, q, r, src, dst):
    mesh = plsc.VectorSubcoreMesh(
        core_axis_name="c", subcore_axis_name="s",
        num_cores=_NC, num_subcores=_NS,
    )
    fn = functools.partial(
        pl.kernel,
        compiler_params=pltpu.CompilerParams(needs_layout_passes=False, use_tc_tiling_on_sc=False),
        out_type=[
            jax.ShapeDtypeStruct((_NC, _NP, _D), jnp.float32),
            jax.ShapeDtypeStruct((_NC, _CR, _D), jnp.float32),
        ],
        mesh=mesh,
        scratch_types=[
            pltpu.VMEM_SHARED((_NP, _D), jnp.float32),
            pltpu.VMEM_SHARED((_CR, _D), jnp.float32),
            pltpu.VMEM((_CHUNK,), jnp.int32),
            pltpu.VMEM((_CHUNK,), jnp.int32),
            pltpu.VMEM((_CHUNK, _D), jnp.float32),
            pltpu.VMEM((_CHUNK, _D), jnp.float32),
            pltpu.VMEM((_CR, _D), jnp.float32),
            pltpu.VMEM((_CR,), jnp.int32),
            pltpu.VMEM((_CHUNK * 17 + 16,), jnp.float32),
            pltpu.VMEM((_CHUNK * 17 + 16,), jnp.float32),
            pltpu.VMEM((_CHUNK,), jnp.float32),
            pltpu.VMEM((_CHUNK,), jnp.float32),
            pltpu.SemaphoreType.DMA,
            pltpu.SemaphoreType.DMA,
            pltpu.SemaphoreType.DMA,
        ],
    )(_sc_edge_body)
    return fn(p, q, r, src, dst)


def _node_body(part_ref, cnt_ref, h_ref, orig_ref, we2_ref, wa_ref, wb_ref,
               wc_ref, wn2_ref, ge_ref, bne_ref, be2_ref, bn1_ref, gn_ref,
               bnn_ref, bn2_ref, gf_ref, bf_ref, out_ref):
    cnt = cnt_ref[0] + cnt_ref[1]
    asum = (part_ref[0] + part_ref[1]) * ge_ref[...] + cnt * bne_ref[...]
    inv = 1.0 / jnp.maximum(cnt, 1.0)
    flag = jnp.minimum(cnt, 1.0)
    aggr = (
        jnp.dot(asum * inv, we2_ref[...], preferred_element_type=jnp.float32)
        + flag * be2_ref[...]
    )
    h = h_ref[...]
    x = (
        jnp.dot(h, wa_ref[...], preferred_element_type=jnp.float32)
        + jnp.dot(aggr, wb_ref[...], preferred_element_type=jnp.float32)
        + jnp.dot(orig_ref[...], wc_ref[...], preferred_element_type=jnp.float32)
        + bn1_ref[...]
    )
    x = jnp.maximum(x, 0.0)
    x = _ln_tc(x, gn_ref[...], bnn_ref[...])
    upd = jnp.dot(x, wn2_ref[...], preferred_element_type=jnp.float32) + bn2_ref[...]
    out = _SKIPW * upd + (1.0 - _SKIPW) * h
    out_ref[...] = _ln_tc(out, gf_ref[...], bf_ref[...])


def _node(part, cnt3, h, orig, we2, wn1a, wn1b, wn1c, wn2, rows):
    nb = 1000
    mat = lambda: pl.BlockSpec((_D, _D), lambda i: (0, 0))
    vec = lambda: pl.BlockSpec((1, _D), lambda i: (0, 0))
    return pl.pallas_call(
        _node_body,
        grid=(_N // nb,),
        in_specs=[
            pl.BlockSpec((_NC, nb, _D), lambda i: (0, i, 0)),
            pl.BlockSpec((_NC, nb, 1), lambda i: (0, i, 0)),
            pl.BlockSpec((nb, _D), lambda i: (i, 0)),
            pl.BlockSpec((nb, _D), lambda i: (i, 0)),
            mat(), mat(), mat(), mat(), mat(),
            vec(), vec(), vec(), vec(), vec(), vec(), vec(), vec(), vec(),
        ],
        out_specs=pl.BlockSpec((nb, _D), lambda i: (i, 0)),
        out_shape=jax.ShapeDtypeStruct((_N, _D), jnp.float32),
    )(part, cnt3, h, orig, we2, wn1a, wn1b, wn1c, wn2, *rows)


def kernel(coords_rec, h_feats_rec, original_receptor_node_features, edge_feat,
           We1, be1, ge, bne, We2, be2, Wn1, bn1, gn, bnn, Wn2, bn2, gf, bf,
           edge_index):
    del coords_rec
    src = edge_index[0].astype(jnp.int32)
    dst = edge_index[1].astype(jnp.int32)
    w1a = We1[:_D]
    w1b = We1[_D:2 * _D]
    w1c = We1[2 * _D:]

    p, q = _pq(h_feats_rec, w1a, w1b)
    r = _redge(edge_feat, w1c, be1.reshape(1, _D))
    part, cnt = _sc_edge(p, q, r, src, dst)
    cnt3 = cnt.reshape(_NC, _NP, 1)

    rows = [v.reshape(1, _D)
            for v in (ge, bne, be2, bn1, gn, bnn, bn2, gf, bf)]
    return _node(part, cnt3, h_feats_rec, original_receptor_node_features,
                 We2, Wn1[:_D], Wn1[_D:2 * _D], Wn1[2 * _D:], Wn2, rows)


# speedup vs baseline: 4.6880x; 1.0710x over previous
import functools

import jax
import jax.numpy as jnp
from jax import lax
from jax.experimental import pallas as pl
from jax.experimental.pallas import tpu as pltpu
from jax.experimental.pallas import tpu_sc as plsc

_N = 10000
_E = 320000
_D = 128
_DE = 16
_SKIPW = 0.5
_EPS = 1e-5

_NC = 2
_NS = 16
_EW = _E // (_NC * _NS)
_CHUNK = 80
_NCH = _EW // _CHUNK
_NP = 10240
_RPT = _NP // _NS
_RCH = _RPT // _CHUNK
_CR = _NP // _D


def _ln_tc(x, g, b):
    mu = jnp.mean(x, axis=-1, keepdims=True)
    var = jnp.mean((x - mu) ** 2, axis=-1, keepdims=True)
    return (x - mu) * lax.rsqrt(var + _EPS) * g + b


def _pq_body(h_ref, wa_ref, wb_ref, p_ref, q_ref):
    h = h_ref[...]
    p_ref[...] = jnp.dot(h, wa_ref[...], preferred_element_type=jnp.float32)
    q_ref[...] = jnp.dot(h, wb_ref[...], preferred_element_type=jnp.float32)


def _pq(h, wa, wb):
    nb = 1000
    return pl.pallas_call(
        _pq_body,
        grid=(_N // nb,),
        in_specs=[
            pl.BlockSpec((nb, _D), lambda i: (i, 0)),
            pl.BlockSpec((_D, _D), lambda i: (0, 0)),
            pl.BlockSpec((_D, _D), lambda i: (0, 0)),
        ],
        out_specs=[
            pl.BlockSpec((nb, _D), lambda i: (i, 0)),
            pl.BlockSpec((nb, _D), lambda i: (i, 0)),
        ],
        out_shape=[
            jax.ShapeDtypeStruct((_N, _D), jnp.float32),
            jax.ShapeDtypeStruct((_N, _D), jnp.float32),
        ],
    )(h, wa, wb)


def _r_body(ef_ref, wc_ref, be1_ref, r_ref):
    r_ref[...] = (
        jnp.dot(ef_ref[...], wc_ref[...], preferred_element_type=jnp.float32)
        + be1_ref[...]
    )


def _redge(ef, wc, be1_row):
    eb = 8000
    return pl.pallas_call(
        _r_body,
        grid=(_E // eb,),
        in_specs=[
            pl.BlockSpec((eb, _DE), lambda i: (i, 0)),
            pl.BlockSpec((_DE, _D), lambda i: (0, 0)),
            pl.BlockSpec((1, _D), lambda i: (0, 0)),
        ],
        out_specs=pl.BlockSpec((eb, _D), lambda i: (i, 0)),
        out_shape=jax.ShapeDtypeStruct((_E, _D), jnp.float32),
    )(ef, wc, be1_row)


def _sc_edge_body(p_hbm, q_hbm, r_hbm, src_hbm, dst_hbm,
                  out_hbm, cnt_hbm, acc, cntacc, src_v, dst_v,
                  r_v, u_v, cnt_v, rowidx_v, st_v, sq_v, mv_v, y_v, dsc_v,
                  sem1, sem2, sem3, sem4):
    cc = lax.axis_index("c")
    ss = lax.axis_index("s")
    zero16 = jnp.zeros((16,), jnp.float32)
    one16 = jnp.full((16,), 1.0, jnp.float32)
    lane = lax.iota(jnp.int32, 16)
    cols = [jnp.full((16,), 16 * j, jnp.int32) + lane for j in range(_D // 16)]

    def _zrow(i, carry):
        rows = jnp.full((16,), i, jnp.int32)
        for j in range(_D // 16):
            plsc.store_scatter(u_v, [rows, cols[j]], zero16)
        return carry

    lax.fori_loop(0, _CHUNK, _zrow, 0)

    def _zcnt(i, carry):
        rows = jnp.full((16,), i, jnp.int32)
        for j in range(_D // 16):
            plsc.store_scatter(cnt_v, [rows, cols[j]], zero16)
        return carry

    lax.fori_loop(0, _CR, _zcnt, 0)
    for k in range(_CR // 16):
        rowidx_v[pl.ds(16 * k, 16)] = jnp.full((16,), 16 * k, jnp.int32) + lane

    nbase = ss * _RPT
    for k in range(_RCH):
        pltpu.sync_copy(u_v, acc.at[pl.ds(nbase + k * _CHUNK, _CHUNK), :])

    @pl.when(ss == 0)
    def _():
        for k in range(_CR // _CHUNK):
            pltpu.sync_copy(u_v, cntacc.at[pl.ds(k * _CHUNK, _CHUNK), :])

    plsc.subcore_barrier()

    ebase = (cc * _NS + ss) * _EW

    def _chunk(ci, carry):
        base = ebase + ci * _CHUNK
        cpr = pltpu.async_copy(r_hbm.at[pl.ds(base, _CHUNK), :], r_v, sem3)
        cpa = pltpu.async_copy(src_hbm.at[pl.ds(base, _CHUNK)], src_v, sem1)
        cpb = pltpu.async_copy(dst_hbm.at[pl.ds(base, _CHUNK)], dst_v, sem2)
        cpa.wait()
        cpb.wait()
        cpr.wait()
        cp1 = pltpu.async_copy(p_hbm.at[src_v], r_v, sem1, add=True)
        cp2 = pltpu.async_copy(q_hbm.at[dst_v], r_v, sem2, add=True)

        for k in range(_CHUNK // 16):
            idx = dst_v[pl.ds(16 * k, 16)]
            plsc.addupdate_scatter(cnt_v, [idx >> 7, idx & 127], one16)
        cp1.wait()
        cp2.wait()

        @plsc.parallel_loop(0, _CHUNK, 1, unroll=2)
        def _pA(e):
            tot = zero16
            sq = zero16
            for j in range(_D // 16):
                t = jnp.maximum(r_v[e, pl.ds(16 * j, 16)], zero16)
                tot = tot + t
                sq = sq + t * t
            st_v[pl.ds(e * 17, 16)] = tot
            sq_v[pl.ds(e * 17, 16)] = sq

        c15 = jnp.full((16,), 1.5, jnp.float32)
        ch = jnp.full((16,), 0.5, jnp.float32)
        for g in range(_CHUNK // 16):
            gidx = (lane + jnp.full((16,), 16 * g, jnp.int32)) * 17
            tot16 = zero16
            sq16 = zero16
            for l in range(16):
                il = gidx + jnp.full((16,), l, jnp.int32)
                tot16 = tot16 + plsc.load_gather(st_v, [il])
                sq16 = sq16 + plsc.load_gather(sq_v, [il])
            m16 = tot16 * (1.0 / _D)
            xv = jnp.maximum(sq16 * (1.0 / _D) - m16 * m16, zero16) + _EPS
            xi = plsc.bitcast(xv, jnp.int32)
            y = plsc.bitcast(jnp.full((16,), 0x5F3759DF, jnp.int32) - (xi >> 1),
                             jnp.float32)
            for _ in range(3):
                y = y * (c15 - ch * xv * y * y)
            mv_v[pl.ds(16 * g, 16)] = m16
            y_v[pl.ds(16 * g, 16)] = y

        @pl.when(ci > 0)
        def _():
            pltpu.make_async_copy(u_v, acc.at[dsc_v], sem4).wait()

        @plsc.parallel_loop(0, _CHUNK, 1, unroll=2)
        def _pC(e):
            eb16 = jnp.full((16,), e, jnp.int32)
            mvb = plsc.load_gather(mv_v, [eb16])
            yb = plsc.load_gather(y_v, [eb16])
            for j in range(_D // 16):
                t = jnp.maximum(r_v[e, pl.ds(16 * j, 16)], zero16)
                u_v[e, pl.ds(16 * j, 16)] = (t - mvb) * yb

        for k in range(_CHUNK // 16):
            dsc_v[pl.ds(16 * k, 16)] = dst_v[pl.ds(16 * k, 16)]
        pltpu.async_copy(u_v, acc.at[dsc_v], sem4, add=True)
        return carry

    lax.fori_loop(0, _NCH, _chunk, 0)
    pltpu.make_async_copy(u_v, acc.at[dsc_v], sem4).wait()

    pltpu.sync_copy(cnt_v, cntacc.at[rowidx_v], add=True)
    plsc.subcore_barrier()

    for k in range(_RCH):
        pltpu.sync_copy(acc.at[pl.ds(nbase + k * _CHUNK, _CHUNK), :], u_v)
        pltpu.sync_copy(u_v, out_hbm.at[cc, pl.ds(nbase + k * _CHUNK, _CHUNK), :])

    @pl.when(ss == 0)
    def _():
        pltpu.sync_copy(cntacc, cnt_hbm.at[cc])


def _sc_edge(p, q, r, src, dst):
    mesh = plsc.VectorSubcoreMesh(
        core_axis_name="c", subcore_axis_name="s",
        num_cores=_NC, num_subcores=_NS,
    )
    fn = functools.partial(
        pl.kernel,
        compiler_params=pltpu.CompilerParams(needs_layout_passes=False, use_tc_tiling_on_sc=False),
        out_type=[
            jax.ShapeDtypeStruct((_NC, _NP, _D), jnp.float32),
            jax.ShapeDtypeStruct((_NC, _CR, _D), jnp.float32),
        ],
        mesh=mesh,
        scratch_types=[
            pltpu.VMEM_SHARED((_NP, _D), jnp.float32),
            pltpu.VMEM_SHARED((_CR, _D), jnp.float32),
            pltpu.VMEM((_CHUNK,), jnp.int32),
            pltpu.VMEM((_CHUNK,), jnp.int32),
            pltpu.VMEM((_CHUNK, _D), jnp.float32),
            pltpu.VMEM((_CHUNK, _D), jnp.float32),
            pltpu.VMEM((_CR, _D), jnp.float32),
            pltpu.VMEM((_CR,), jnp.int32),
            pltpu.VMEM((_CHUNK * 17 + 16,), jnp.float32),
            pltpu.VMEM((_CHUNK * 17 + 16,), jnp.float32),
            pltpu.VMEM((_CHUNK,), jnp.float32),
            pltpu.VMEM((_CHUNK,), jnp.float32),
            pltpu.VMEM((_CHUNK,), jnp.int32),
            pltpu.SemaphoreType.DMA,
            pltpu.SemaphoreType.DMA,
            pltpu.SemaphoreType.DMA,
            pltpu.SemaphoreType.DMA,
        ],
    )(_sc_edge_body)
    return fn(p, q, r, src, dst)


def _node_body(part_ref, cnt_ref, h_ref, orig_ref, we2_ref, wa_ref, wb_ref,
               wc_ref, wn2_ref, ge_ref, bne_ref, be2_ref, bn1_ref, gn_ref,
               bnn_ref, bn2_ref, gf_ref, bf_ref, out_ref):
    cnt = cnt_ref[0] + cnt_ref[1]
    asum = (part_ref[0] + part_ref[1]) * ge_ref[...] + cnt * bne_ref[...]
    inv = 1.0 / jnp.maximum(cnt, 1.0)
    flag = jnp.minimum(cnt, 1.0)
    aggr = (
        jnp.dot(asum * inv, we2_ref[...], preferred_element_type=jnp.float32)
        + flag * be2_ref[...]
    )
    h = h_ref[...]
    x = (
        jnp.dot(h, wa_ref[...], preferred_element_type=jnp.float32)
        + jnp.dot(aggr, wb_ref[...], preferred_element_type=jnp.float32)
        + jnp.dot(orig_ref[...], wc_ref[...], preferred_element_type=jnp.float32)
        + bn1_ref[...]
    )
    x = jnp.maximum(x, 0.0)
    x = _ln_tc(x, gn_ref[...], bnn_ref[...])
    upd = jnp.dot(x, wn2_ref[...], preferred_element_type=jnp.float32) + bn2_ref[...]
    out = _SKIPW * upd + (1.0 - _SKIPW) * h
    out_ref[...] = _ln_tc(out, gf_ref[...], bf_ref[...])


def _node(part, cnt3, h, orig, we2, wn1a, wn1b, wn1c, wn2, rows):
    nb = 1000
    mat = lambda: pl.BlockSpec((_D, _D), lambda i: (0, 0))
    vec = lambda: pl.BlockSpec((1, _D), lambda i: (0, 0))
    return pl.pallas_call(
        _node_body,
        grid=(_N // nb,),
        in_specs=[
            pl.BlockSpec((_NC, nb, _D), lambda i: (0, i, 0)),
            pl.BlockSpec((_NC, nb, 1), lambda i: (0, i, 0)),
            pl.BlockSpec((nb, _D), lambda i: (i, 0)),
            pl.BlockSpec((nb, _D), lambda i: (i, 0)),
            mat(), mat(), mat(), mat(), mat(),
            vec(), vec(), vec(), vec(), vec(), vec(), vec(), vec(), vec(),
        ],
        out_specs=pl.BlockSpec((nb, _D), lambda i: (i, 0)),
        out_shape=jax.ShapeDtypeStruct((_N, _D), jnp.float32),
    )(part, cnt3, h, orig, we2, wn1a, wn1b, wn1c, wn2, *rows)


def kernel(coords_rec, h_feats_rec, original_receptor_node_features, edge_feat,
           We1, be1, ge, bne, We2, be2, Wn1, bn1, gn, bnn, Wn2, bn2, gf, bf,
           edge_index):
    del coords_rec
    src = edge_index[0].astype(jnp.int32)
    dst = edge_index[1].astype(jnp.int32)
    w1a = We1[:_D]
    w1b = We1[_D:2 * _D]
    w1c = We1[2 * _D:]

    p, q = _pq(h_feats_rec, w1a, w1b)
    r = _redge(edge_feat, w1c, be1.reshape(1, _D))
    part, cnt = _sc_edge(p, q, r, src, dst)
    cnt3 = cnt.reshape(_NC, _NP, 1)

    rows = [v.reshape(1, _D)
            for v in (ge, bne, be2, bn1, gn, bnn, bn2, gf, bf)]
    return _node(part, cnt3, h_feats_rec, original_receptor_node_features,
                 We2, Wn1[:_D], Wn1[_D:2 * _D], Wn1[2 * _D:], Wn2, rows)
